# Initial kernel scaffold; baseline (speedup 1.0000x reference)
import jax
import jax.numpy as jnp
from jax import lax
from jax.experimental import pallas as pl
from jax.experimental.pallas import tpu as pltpu
from jax.experimental.pallas import tpu_sc as plsc

N, E, D, HID, H, C, L = 10000, 160000, 256, 256, 4, 64, 5
NC, NS, LANES = 2, 16, 16
BLK = 125
NB = E // (NS * BLK)
ACCW = 144
ROWS = 1000
HALF = HID // 2


def _sc_edge_body(xl_hbm, xr_hbm, sadj_hbm, dadj_hbm, dorg_hbm, att_hbm,
                  out_hbm, sadj_v, dadj_v, dorg_v, att_v, xlg, xrg, wv,
                  acc_sh, sem0, sem1):
    c = lax.axis_index("c")
    t = lax.axis_index("s")
    pltpu.sync_copy(sadj_hbm.at[c, t], sadj_v)
    pltpu.sync_copy(dadj_hbm.at[c, t], dadj_v)
    pltpu.sync_copy(dorg_hbm.at[t], dorg_v)
    pltpu.sync_copy(att_hbm.at[pl.ds(2 * c, 2)], att_v)

    zeros = jnp.zeros((LANES,), jnp.float32)

    def zero_row(e, carry):
        for k in range(ACCW // LANES):
            wv[e, pl.ds(k * LANES, LANES)] = zeros
        return carry

    lax.fori_loop(0, BLK, zero_row, 0)
    rpt = N // NS
    for j in range(rpt // BLK):
        pltpu.sync_copy(wv, acc_sh.at[pl.ds(t * rpt + j * BLK, BLK)])
    plsc.subcore_barrier()

    att_r = ([att_v[0, pl.ds(k * LANES, LANES)] for k in range(4)]
             + [att_v[1, pl.ds(k * LANES, LANES)] for k in range(4)])
    lane = lax.iota(jnp.int32, LANES)

    def block(b, carry):
        cp0 = pltpu.async_copy(xl_hbm.at[sadj_v.at[b]], xlg, sem0)
        cp1 = pltpu.async_copy(xr_hbm.at[dadj_v.at[b]], xrg, sem1)
        cp0.wait()
        cp1.wait()

        def edge(e, ecarry):
            xlv = [xlg[e, pl.ds(k * LANES, LANES)] for k in range(8)]
            xrv = [xrg[e, pl.ds(k * LANES, LANES)] for k in range(8)]
            p = []
            for k in range(8):
                mm = xlv[k] + xrv[k]
                mm = jnp.maximum(mm, 0.2 * mm)
                p.append(mm * att_r[k])
            s0 = jnp.sum(p[0] + p[1] + p[2] + p[3])
            s1 = jnp.sum(p[4] + p[5] + p[6] + p[7])
            e0 = jnp.exp(jnp.broadcast_to(s0, (LANES,)))
            e1 = jnp.exp(jnp.broadcast_to(s1, (LANES,)))
            for k in range(4):
                wv[e, pl.ds(k * LANES, LANES)] = xlv[k] * e0
            for k in range(4, 8):
                wv[e, pl.ds(k * LANES, LANES)] = xlv[k] * e1
            den = jnp.where(lane == 0, e0,
                            jnp.where(lane == 1, e1, jnp.zeros_like(e0)))
            wv[e, pl.ds(8 * LANES, LANES)] = den
            return ecarry

        lax.fori_loop(0, BLK, edge, 0, unroll=2)
        pltpu.sync_copy(wv, acc_sh.at[dorg_v.at[b]], add=True)
        return carry

    lax.fori_loop(0, NB, block, 0)
    plsc.subcore_barrier()
    pltpu.sync_copy(acc_sh.at[pl.ds(t * rpt, rpt)],
                    out_hbm.at[c, pl.ds(t * rpt, rpt)])


def _sc_edge(xl2n, xr2n, sadj, dadj, dorg, att_l):
    mesh = plsc.VectorSubcoreMesh(core_axis_name="c", subcore_axis_name="s")
    return pl.kernel(
        _sc_edge_body,
        out_type=jax.ShapeDtypeStruct((NC, N, ACCW), jnp.float32),
        mesh=mesh,
        scratch_types=[
            pltpu.VMEM((NB, BLK), jnp.int32),
            pltpu.VMEM((NB, BLK), jnp.int32),
            pltpu.VMEM((NB, BLK), jnp.int32),
            pltpu.VMEM((H, C), jnp.float32),
            pltpu.VMEM((BLK, HALF), jnp.float32),
            pltpu.VMEM((BLK, HALF), jnp.float32),
            pltpu.VMEM((BLK, ACCW), jnp.float32),
            pltpu.VMEM_SHARED((N, ACCW), jnp.float32),
            pltpu.SemaphoreType.DMA,
            pltpu.SemaphoreType.DMA,
        ],
    )(xl2n, xr2n, sadj, dadj, dorg, att_l)


def _ln_rows(t0, t1, lg, lb):
    s = jnp.sum(t0, -1, keepdims=True) + jnp.sum(t1, -1, keepdims=True)
    mu = s / HID
    v = (jnp.sum((t0 - mu) ** 2, -1, keepdims=True)
         + jnp.sum((t1 - mu) ** 2, -1, keepdims=True))
    inv = 1.0 / jnp.sqrt(v / HID + 1e-5)
    y0 = (t0 - mu) * inv * lg[0:1, :] + lb[0:1, :]
    y1 = (t1 - mu) * inv * lg[1:2, :] + lb[1:2, :]
    return y0, y1


def _k0_body(feat, mask, pW, pb, pg, pbeta, semb, Wl0, bl0, Wr0, br0,
             x_out, xl_out, xr_out):
    f = feat[...]
    z = jnp.dot(f, pW[...], preferred_element_type=jnp.float32) + pb[...]
    mu = jnp.mean(z, axis=-1, keepdims=True)
    var = jnp.mean((z - mu) ** 2, axis=-1, keepdims=True)
    xn = (z - mu) / jnp.sqrt(var + 1e-5) * pg[...] + pbeta[...]
    m = mask[...]
    emb = jnp.where(m == 0, semb[0:1, :], semb[1:2, :])
    x = xn + emb
    xl = jnp.dot(x, Wl0[...], preferred_element_type=jnp.float32) + bl0[...]
    xr = jnp.dot(x, Wr0[...], preferred_element_type=jnp.float32) + br0[...]
    x_out[0] = x[:, :HALF]
    x_out[1] = x[:, HALF:]
    xl_out[0] = xl[:, :HALF]
    xl_out[1] = xl[:, HALF:]
    xr_out[0] = xr[:, :HALF]
    xr_out[1] = xr[:, HALF:]


def _combine_halves(acc, x, xl, xr, attbd, hm, cb):
    ts = []
    for q in (0, 1):
        xlq = xl[q]
        xrq = xr[q]
        mm = xlq + xrq
        mm = jnp.maximum(mm, 0.2 * mm)
        a2 = jnp.dot(mm, attbd[q], preferred_element_type=jnp.float32)
        exs = jnp.exp(a2)
        den2 = acc[q][:, 8 * LANES:8 * LANES + 2] + exs
        numer = (acc[q][:, :HALF]
                 + jnp.dot(exs, hm[...], preferred_element_type=jnp.float32)
                 * xlq)
        hq = numer / jnp.dot(den2, hm[...],
                             preferred_element_type=jnp.float32)
        ts.append(hq + cb[q:q + 1, :] + x[q])
    return ts


def _tc_mid_body(acc, x, xl, xr, attbd, hm, cb, lg, lb, Wln, bln, Wrn, brn,
                 x_out, xl_out, xr_out):
    t0, t1 = _combine_halves(acc, x, xl, xr, attbd, hm, cb)
    y0, y1 = _ln_rows(t0, t1, lg, lb)
    y = jnp.concatenate([y0, y1], axis=1)
    x_out[0] = y0
    x_out[1] = y1
    xln = jnp.dot(y, Wln[...], preferred_element_type=jnp.float32) + bln[...]
    xrn = jnp.dot(y, Wrn[...], preferred_element_type=jnp.float32) + brn[...]
    xl_out[0] = xln[:, :HALF]
    xl_out[1] = xln[:, HALF:]
    xr_out[0] = xrn[:, :HALF]
    xr_out[1] = xrn[:, HALF:]


def _tc_fin_body(acc, x, xl, xr, attbd, hm, cb, lg, lb, hW1, hb1, hW2, hb2,
                 o_out):
    t0, t1 = _combine_halves(acc, x, xl, xr, attbd, hm, cb)
    y0, y1 = _ln_rows(t0, t1, lg, lb)
    y = jnp.concatenate([y0, y1], axis=1)
    hh = jnp.dot(y, hW1[...], preferred_element_type=jnp.float32) + hb1[...]
    g = 0.5 * hh * (1.0 + lax.erf(hh / jnp.sqrt(jnp.float32(2.0))))
    o_out[...] = (jnp.dot(g, hW2[...], preferred_element_type=jnp.float32)
                  + hb2[...])


def _full(shape):
    return pl.BlockSpec(shape, lambda i: tuple(0 for _ in shape))


def _rows(shape3):
    return pl.BlockSpec(shape3, lambda i: (0, i, 0))


_GRID = N // ROWS

_k0_call = pl.pallas_call(
    _k0_body,
    grid=(_GRID,),
    in_specs=[
        pl.BlockSpec((ROWS, D), lambda i: (i, 0)),
        pl.BlockSpec((ROWS, 1), lambda i: (i, 0)),
        _full((D, HID)), _full((1, HID)), _full((1, HID)), _full((1, HID)),
        _full((2, HID)),
        _full((HID, HID)), _full((1, HID)),
        _full((HID, HID)), _full((1, HID)),
    ],
    out_specs=[_rows((2, ROWS, HALF))] * 3,
    out_shape=[jax.ShapeDtypeStruct((2, N, HALF), jnp.float32)] * 3,
)

_tc_mid_call = pl.pallas_call(
    _tc_mid_body,
    grid=(_GRID,),
    in_specs=[
        _rows((2, ROWS, ACCW)),
        _rows((2, ROWS, HALF)), _rows((2, ROWS, HALF)), _rows((2, ROWS, HALF)),
        _full((2, HALF, 2)), _full((2, HALF)),
        _full((2, HALF)), _full((2, HALF)), _full((2, HALF)),
        _full((HID, HID)), _full((1, HID)),
        _full((HID, HID)), _full((1, HID)),
    ],
    out_specs=[_rows((2, ROWS, HALF))] * 3,
    out_shape=[jax.ShapeDtypeStruct((2, N, HALF), jnp.float32)] * 3,
)

_tc_fin_call = pl.pallas_call(
    _tc_fin_body,
    grid=(_GRID,),
    in_specs=[
        _rows((2, ROWS, ACCW)),
        _rows((2, ROWS, HALF)), _rows((2, ROWS, HALF)), _rows((2, ROWS, HALF)),
        _full((2, HALF, 2)), _full((2, HALF)),
        _full((2, HALF)), _full((2, HALF)), _full((2, HALF)),
        _full((HID, 128)), _full((1, 128)),
        _full((128, 1)), _full((1, 1)),
    ],
    out_specs=pl.BlockSpec((ROWS, 1), lambda i: (i, 0)),
    out_shape=jax.ShapeDtypeStruct((N, 1), jnp.float32),
)


def kernel(features, edge_index, scale_mask, proj_W, proj_b, proj_g,
           proj_beta, scale_embed, Wl, bl, Wr, br, att, conv_b, ln_g, ln_b,
           head_W1, head_b1, head_W2, head_b2):
    f32 = jnp.float32
    ei = edge_index.astype(jnp.int32)
    src = ei[0].reshape(NS, NB, BLK)
    dst = ei[1].reshape(NS, NB, BLK)
    sadj = jnp.stack([src, src + N])
    dadj = jnp.stack([dst, dst + N])
    mask2 = scale_mask.astype(jnp.int32).reshape(N, 1)

    def row2(v):
        return v.astype(f32).reshape(1, -1)

    def halves(v):
        return v.astype(f32).reshape(2, HALF)

    hm = jnp.concatenate(
        [jnp.concatenate([jnp.ones((1, C), f32), jnp.zeros((1, C), f32)], 1),
         jnp.concatenate([jnp.zeros((1, C), f32), jnp.ones((1, C), f32)], 1)],
        axis=0)

    x, xl, xr = _k0_call(features.astype(f32), mask2, proj_W.astype(f32),
                         row2(proj_b), row2(proj_g), row2(proj_beta),
                         scale_embed.astype(f32), Wl[0].astype(f32),
                         row2(bl[0]), Wr[0].astype(f32), row2(br[0]))

    out = None
    zc = jnp.zeros((2, C), f32)
    for l in range(L):
        att_l = att[l].astype(f32)
        a_even = jnp.stack([att_l[0], att_l[2]])
        a_odd = jnp.stack([att_l[1], att_l[3]])
        colA = jnp.concatenate([a_even, zc], axis=1)
        colB = jnp.concatenate([zc, a_odd], axis=1)
        attbd = jnp.stack([colA, colB], axis=-1)

        acc = _sc_edge(xl.reshape(2 * N, HALF), xr.reshape(2 * N, HALF),
                       sadj, dadj, dst, att_l)

        if l < L - 1:
            x, xl, xr = _tc_mid_call(
                acc, x, xl, xr, attbd, hm, halves(conv_b[l]),
                halves(ln_g[l]), halves(ln_b[l]),
                Wl[l + 1].astype(f32), row2(bl[l + 1]),
                Wr[l + 1].astype(f32), row2(br[l + 1]))
        else:
            out = _tc_fin_call(
                acc, x, xl, xr, attbd, hm, halves(conv_b[l]),
                halves(ln_g[l]), halves(ln_b[l]),
                head_W1.astype(f32), row2(head_b1),
                head_W2.astype(f32), head_b2.astype(f32).reshape(1, 1))
    return out[:, 0]


# speedup vs baseline: 20.5969x; 20.5969x over previous
import jax
import jax.numpy as jnp
from jax import lax
from jax.experimental import pallas as pl
from jax.experimental.pallas import tpu as pltpu
from jax.experimental.pallas import tpu_sc as plsc

N, E, D, HID, H, C, L = 10000, 160000, 256, 256, 4, 64, 5
NC, NS, LANES = 2, 16, 16
BLK = 80
NB = E // (NS * BLK)
ACCW = 144
ROWS = 1000
HALF = HID // 2


def _sc_edge_body(xl_hbm, xr_hbm, tab_hbm, att_hbm,
                  out_hbm, tab3, att_v, xlg, xrg, wv,
                  acc_sh, sem0, sem1):
    c = lax.axis_index("c")
    t = lax.axis_index("s")
    pltpu.sync_copy(att_hbm.at[pl.ds(2 * c, 2)], att_v)

    zeros = jnp.zeros((LANES,), jnp.float32)

    def zero_row(e, carry):
        for k in range(ACCW // LANES):
            wv[e, pl.ds(k * LANES, LANES)] = zeros
        return carry

    lax.fori_loop(0, BLK, zero_row, 0)
    rpt = N // NS
    nz = rpt // BLK
    for j in range(nz):
        pltpu.sync_copy(wv, acc_sh.at[pl.ds(t * rpt + j * BLK, BLK)])
    rem = rpt - nz * BLK
    if rem:
        pltpu.sync_copy(wv.at[pl.ds(0, rem)],
                        acc_sh.at[pl.ds(t * rpt + nz * BLK, rem)])
    plsc.subcore_barrier()

    att_r = ([att_v[0, pl.ds(k * LANES, LANES)] for k in range(4)]
             + [att_v[1, pl.ds(k * LANES, LANES)] for k in range(4)])
    lane = lax.iota(jnp.int32, LANES)

    def block(b, carry):
        pltpu.sync_copy(tab_hbm.at[c, t, b], tab3)
        cp0 = pltpu.async_copy(xl_hbm.at[tab3.at[0]], xlg, sem0)
        cp1 = pltpu.async_copy(xr_hbm.at[tab3.at[1]], xrg, sem1)
        cp0.wait()
        cp1.wait()

        def edge(e, ecarry):
            xlv = [xlg[e, pl.ds(k * LANES, LANES)] for k in range(8)]
            xrv = [xrg[e, pl.ds(k * LANES, LANES)] for k in range(8)]
            p = []
            for k in range(8):
                mm = xlv[k] + xrv[k]
                mm = jnp.maximum(mm, 0.2 * mm)
                p.append(mm * att_r[k])
            s0 = jnp.sum(p[0] + p[1] + p[2] + p[3])
            s1 = jnp.sum(p[4] + p[5] + p[6] + p[7])
            e0 = jnp.exp(jnp.broadcast_to(s0, (LANES,)))
            e1 = jnp.exp(jnp.broadcast_to(s1, (LANES,)))
            for k in range(4):
                wv[e, pl.ds(k * LANES, LANES)] = xlv[k] * e0
            for k in range(4, 8):
                wv[e, pl.ds(k * LANES, LANES)] = xlv[k] * e1
            den = jnp.where(lane == 0, e0,
                            jnp.where(lane == 1, e1, jnp.zeros_like(e0)))
            wv[e, pl.ds(8 * LANES, LANES)] = den
            return ecarry

        lax.fori_loop(0, BLK, edge, 0, unroll=2)
        pltpu.sync_copy(wv, acc_sh.at[tab3.at[2]], add=True)
        return carry

    lax.fori_loop(0, NB, block, 0)
    plsc.subcore_barrier()
    pltpu.sync_copy(acc_sh.at[pl.ds(t * rpt, rpt)],
                    out_hbm.at[c, pl.ds(t * rpt, rpt)])


def _sc_edge(xl2n, xr2n, tab, att_l):
    mesh = plsc.VectorSubcoreMesh(core_axis_name="c", subcore_axis_name="s")
    return pl.kernel(
        _sc_edge_body,
        out_type=jax.ShapeDtypeStruct((NC, N, ACCW), jnp.float32),
        mesh=mesh,
        compiler_params=pltpu.CompilerParams(use_tc_tiling_on_sc=False,
                                             needs_layout_passes=False),
        scratch_types=[
            pltpu.VMEM((3, BLK), jnp.int32),
            pltpu.VMEM((2, C), jnp.float32),
            pltpu.VMEM((BLK, HALF), jnp.float32),
            pltpu.VMEM((BLK, HALF), jnp.float32),
            pltpu.VMEM((BLK, ACCW), jnp.float32),
            pltpu.VMEM_SHARED((N, ACCW), jnp.float32),
            pltpu.SemaphoreType.DMA,
            pltpu.SemaphoreType.DMA,
        ],
    )(xl2n, xr2n, tab, att_l)


def _ln_rows(t0, t1, lg, lb):
    s = jnp.sum(t0, -1, keepdims=True) + jnp.sum(t1, -1, keepdims=True)
    mu = s / HID
    v = (jnp.sum((t0 - mu) ** 2, -1, keepdims=True)
         + jnp.sum((t1 - mu) ** 2, -1, keepdims=True))
    inv = 1.0 / jnp.sqrt(v / HID + 1e-5)
    y0 = (t0 - mu) * inv * lg[0:1, :] + lb[0:1, :]
    y1 = (t1 - mu) * inv * lg[1:2, :] + lb[1:2, :]
    return y0, y1


def _k0_body(feat, mask, pW, pb, pg, pbeta, semb, Wl0, bl0, Wr0, br0,
             x_out, xl_out, xr_out):
    f = feat[...]
    z = jnp.dot(f, pW[...], preferred_element_type=jnp.float32) + pb[...]
    mu = jnp.mean(z, axis=-1, keepdims=True)
    var = jnp.mean((z - mu) ** 2, axis=-1, keepdims=True)
    xn = (z - mu) / jnp.sqrt(var + 1e-5) * pg[...] + pbeta[...]
    m = mask[...]
    emb = jnp.where(m == 0, semb[0:1, :], semb[1:2, :])
    x = xn + emb
    xl = jnp.dot(x, Wl0[...], preferred_element_type=jnp.float32) + bl0[...]
    xr = jnp.dot(x, Wr0[...], preferred_element_type=jnp.float32) + br0[...]
    x_out[0] = x[:, :HALF]
    x_out[1] = x[:, HALF:]
    xl_out[0] = xl[:, :HALF]
    xl_out[1] = xl[:, HALF:]
    xr_out[0] = xr[:, :HALF]
    xr_out[1] = xr[:, HALF:]


def _combine_halves(acc, x, xl, xr, attbd, hm, cb):
    ts = []
    for q in (0, 1):
        xlq = xl[q]
        xrq = xr[q]
        mm = xlq + xrq
        mm = jnp.maximum(mm, 0.2 * mm)
        a2 = jnp.dot(mm, attbd[q], preferred_element_type=jnp.float32)
        exs = jnp.exp(a2)
        den2 = acc[q][:, 8 * LANES:8 * LANES + 2] + exs
        numer = (acc[q][:, :HALF]
                 + jnp.dot(exs, hm[...], preferred_element_type=jnp.float32)
                 * xlq)
        hq = numer / jnp.dot(den2, hm[...],
                             preferred_element_type=jnp.float32)
        ts.append(hq + cb[q:q + 1, :] + x[q])
    return ts


def _tc_mid_body(acc, x, xl, xr, attbd, hm, cb, lg, lb, Wln, bln, Wrn, brn,
                 x_out, xl_out, xr_out):
    t0, t1 = _combine_halves(acc, x, xl, xr, attbd, hm, cb)
    y0, y1 = _ln_rows(t0, t1, lg, lb)
    y = jnp.concatenate([y0, y1], axis=1)
    x_out[0] = y0
    x_out[1] = y1
    xln = jnp.dot(y, Wln[...], preferred_element_type=jnp.float32) + bln[...]
    xrn = jnp.dot(y, Wrn[...], preferred_element_type=jnp.float32) + brn[...]
    xl_out[0] = xln[:, :HALF]
    xl_out[1] = xln[:, HALF:]
    xr_out[0] = xrn[:, :HALF]
    xr_out[1] = xrn[:, HALF:]


def _tc_fin_body(acc, x, xl, xr, attbd, hm, cb, lg, lb, hW1, hb1, hW2, hb2,
                 o_out):
    t0, t1 = _combine_halves(acc, x, xl, xr, attbd, hm, cb)
    y0, y1 = _ln_rows(t0, t1, lg, lb)
    y = jnp.concatenate([y0, y1], axis=1)
    hh = jnp.dot(y, hW1[...], preferred_element_type=jnp.float32) + hb1[...]
    g = 0.5 * hh * (1.0 + lax.erf(hh / jnp.sqrt(jnp.float32(2.0))))
    o_out[...] = (jnp.dot(g, hW2[...], preferred_element_type=jnp.float32)
                  + hb2[...])


def _full(shape):
    return pl.BlockSpec(shape, lambda i: tuple(0 for _ in shape))


def _rows(shape3):
    return pl.BlockSpec(shape3, lambda i: (0, i, 0))


_GRID = N // ROWS

_k0_call = pl.pallas_call(
    _k0_body,
    grid=(_GRID,),
    in_specs=[
        pl.BlockSpec((ROWS, D), lambda i: (i, 0)),
        pl.BlockSpec((ROWS, 1), lambda i: (i, 0)),
        _full((D, HID)), _full((1, HID)), _full((1, HID)), _full((1, HID)),
        _full((2, HID)),
        _full((HID, HID)), _full((1, HID)),
        _full((HID, HID)), _full((1, HID)),
    ],
    out_specs=[_rows((2, ROWS, HALF))] * 3,
    out_shape=[jax.ShapeDtypeStruct((2, N, HALF), jnp.float32)] * 3,
)

_tc_mid_call = pl.pallas_call(
    _tc_mid_body,
    grid=(_GRID,),
    in_specs=[
        _rows((2, ROWS, ACCW)),
        _rows((2, ROWS, HALF)), _rows((2, ROWS, HALF)), _rows((2, ROWS, HALF)),
        _full((2, HALF, 2)), _full((2, HALF)),
        _full((2, HALF)), _full((2, HALF)), _full((2, HALF)),
        _full((HID, HID)), _full((1, HID)),
        _full((HID, HID)), _full((1, HID)),
    ],
    out_specs=[_rows((2, ROWS, HALF))] * 3,
    out_shape=[jax.ShapeDtypeStruct((2, N, HALF), jnp.float32)] * 3,
)

_tc_fin_call = pl.pallas_call(
    _tc_fin_body,
    grid=(_GRID,),
    in_specs=[
        _rows((2, ROWS, ACCW)),
        _rows((2, ROWS, HALF)), _rows((2, ROWS, HALF)), _rows((2, ROWS, HALF)),
        _full((2, HALF, 2)), _full((2, HALF)),
        _full((2, HALF)), _full((2, HALF)), _full((2, HALF)),
        _full((HID, 128)), _full((1, 128)),
        _full((128, 1)), _full((1, 1)),
    ],
    out_specs=pl.BlockSpec((ROWS, 1), lambda i: (i, 0)),
    out_shape=jax.ShapeDtypeStruct((N, 1), jnp.float32),
)


def kernel(features, edge_index, scale_mask, proj_W, proj_b, proj_g,
           proj_beta, scale_embed, Wl, bl, Wr, br, att, conv_b, ln_g, ln_b,
           head_W1, head_b1, head_W2, head_b2):
    f32 = jnp.float32
    ei = edge_index.astype(jnp.int32)
    src = ei[0].reshape(NS, NB, BLK)
    dst = ei[1].reshape(NS, NB, BLK)
    tab = jnp.stack([
        jnp.stack([src + c * N, dst + c * N, dst], axis=2)
        for c in range(NC)])
    mask2 = scale_mask.astype(jnp.int32).reshape(N, 1)

    def row2(v):
        return v.astype(f32).reshape(1, -1)

    def halves(v):
        return v.astype(f32).reshape(2, HALF)

    hm = jnp.concatenate(
        [jnp.concatenate([jnp.ones((1, C), f32), jnp.zeros((1, C), f32)], 1),
         jnp.concatenate([jnp.zeros((1, C), f32), jnp.ones((1, C), f32)], 1)],
        axis=0)

    x, xl, xr = _k0_call(features.astype(f32), mask2, proj_W.astype(f32),
                         row2(proj_b), row2(proj_g), row2(proj_beta),
                         scale_embed.astype(f32), Wl[0].astype(f32),
                         row2(bl[0]), Wr[0].astype(f32), row2(br[0]))

    out = None
    zc = jnp.zeros((2, C), f32)
    for l in range(L):
        att_l = att[l].astype(f32)
        a_even = jnp.stack([att_l[0], att_l[2]])
        a_odd = jnp.stack([att_l[1], att_l[3]])
        colA = jnp.concatenate([a_even, zc], axis=1)
        colB = jnp.concatenate([zc, a_odd], axis=1)
        attbd = jnp.stack([colA, colB], axis=-1)

        acc = _sc_edge(xl.reshape(2 * N, HALF), xr.reshape(2 * N, HALF),
                       tab, att_l)

        if l < L - 1:
            x, xl, xr = _tc_mid_call(
                acc, x, xl, xr, attbd, hm, halves(conv_b[l]),
                halves(ln_g[l]), halves(ln_b[l]),
                Wl[l + 1].astype(f32), row2(bl[l + 1]),
                Wr[l + 1].astype(f32), row2(br[l + 1]))
        else:
            out = _tc_fin_call(
                acc, x, xl, xr, attbd, hm, halves(conv_b[l]),
                halves(ln_g[l]), halves(ln_b[l]),
                head_W1.astype(f32), row2(head_b1),
                head_W2.astype(f32), head_b2.astype(f32).reshape(1, 1))
    return out[:, 0]


# speedup vs baseline: 28.6247x; 1.3898x over previous
import jax
import jax.numpy as jnp
from jax import lax
from jax.experimental import pallas as pl
from jax.experimental.pallas import tpu as pltpu
from jax.experimental.pallas import tpu_sc as plsc

N, E, D, HID, H, C, L = 10000, 160000, 256, 256, 4, 64, 5
NC, NS, LANES = 2, 16, 16
BLK = 50
NB = E // (NS * BLK)
G = 20
ACCW = 144
ROWS = 1000
HALF = HID // 2


def _sc_edge_body(xl_hbm, xr_hbm, tab_hbm, att_hbm,
                  out_hbm, tabg, att_v, xlg, xrg, wv,
                  acc_sh, sem0, sem1):
    c = lax.axis_index("c")
    t = lax.axis_index("s")
    pltpu.sync_copy(att_hbm.at[pl.ds(2 * c, 2)], att_v)

    zeros = jnp.zeros((LANES,), jnp.float32)

    def zero_row(e, carry):
        for k in range(ACCW // LANES):
            wv[e, pl.ds(k * LANES, LANES)] = zeros
        return carry

    lax.fori_loop(0, BLK, zero_row, 0)
    rpt = N // NS
    nz = rpt // BLK
    for j in range(nz):
        pltpu.sync_copy(wv, acc_sh.at[pl.ds(t * rpt + j * BLK, BLK)])
    rem = rpt - nz * BLK
    if rem:
        pltpu.sync_copy(wv.at[pl.ds(0, rem)],
                        acc_sh.at[pl.ds(t * rpt + nz * BLK, rem)])
    plsc.subcore_barrier()

    att_r = ([att_v[0, pl.ds(k * LANES, LANES)] for k in range(4)]
             + [att_v[1, pl.ds(k * LANES, LANES)] for k in range(4)])
    lane = lax.iota(jnp.int32, LANES)

    def compute_block(slot):
        def edge(e, ecarry):
            xlv = [xlg[slot, e, pl.ds(k * LANES, LANES)] for k in range(8)]
            xrv = [xrg[slot, e, pl.ds(k * LANES, LANES)] for k in range(8)]
            p = []
            for k in range(8):
                mm = xlv[k] + xrv[k]
                mm = jnp.maximum(mm, 0.2 * mm)
                p.append(mm * att_r[k])
            s0 = jnp.sum(p[0] + p[1] + p[2] + p[3])
            s1 = jnp.sum(p[4] + p[5] + p[6] + p[7])
            e0 = jnp.exp(jnp.broadcast_to(s0, (LANES,)))
            e1 = jnp.exp(jnp.broadcast_to(s1, (LANES,)))
            for k in range(4):
                wv[e, pl.ds(k * LANES, LANES)] = xlv[k] * e0
            for k in range(4, 8):
                wv[e, pl.ds(k * LANES, LANES)] = xlv[k] * e1
            den = jnp.where(lane == 0, e0,
                            jnp.where(lane == 1, e1, jnp.zeros_like(e0)))
            wv[e, pl.ds(8 * LANES, LANES)] = den
            return ecarry

        lax.fori_loop(0, BLK, edge, 0, unroll=2)

    def issue(b, slot):
        gs = lax.rem(lax.div(b, G), 2)
        r = lax.rem(b, G)
        cp0 = pltpu.async_copy(xl_hbm.at[tabg.at[gs, r, 0]],
                               xlg.at[slot], sem0)
        cp1 = pltpu.async_copy(xr_hbm.at[tabg.at[gs, r, 1]],
                               xrg.at[slot], sem1)
        return cp0, cp1

    def scatter(b):
        gs = lax.rem(lax.div(b, G), 2)
        r = lax.rem(b, G)
        pltpu.sync_copy(wv, acc_sh.at[tabg.at[gs, r, 2]], add=True)

    pltpu.sync_copy(tab_hbm.at[c, t, pl.ds(0, G)], tabg.at[0])
    issue(0, 0)

    def pair(j, carry):
        b0 = 2 * j
        b1 = b0 + 1
        b2 = b0 + 2
        issue(b1, 1)
        pltpu.make_async_copy(xl_hbm.at[tabg.at[0, 0, 0]], xlg.at[0],
                              sem0).wait()
        pltpu.make_async_copy(xr_hbm.at[tabg.at[0, 0, 1]], xrg.at[0],
                              sem1).wait()
        compute_block(0)
        scatter(b0)

        @pl.when(jnp.logical_and(lax.rem(b2, G) == 0, b2 < NB))
        def _():
            gs2 = lax.rem(lax.div(b2, G), 2)
            pltpu.sync_copy(tab_hbm.at[c, t, pl.ds(b2, G)], tabg.at[gs2])

        @pl.when(b2 < NB)
        def _():
            issue(b2, 0)
        pltpu.make_async_copy(xl_hbm.at[tabg.at[0, 0, 0]], xlg.at[1],
                              sem0).wait()
        pltpu.make_async_copy(xr_hbm.at[tabg.at[0, 0, 1]], xrg.at[1],
                              sem1).wait()
        compute_block(1)
        scatter(b1)
        return carry

    lax.fori_loop(0, NB // 2, pair, 0)
    plsc.subcore_barrier()
    pltpu.sync_copy(acc_sh.at[pl.ds(t * rpt, rpt)],
                    out_hbm.at[c, pl.ds(t * rpt, rpt)])


def _sc_edge(xl2n, xr2n, tab, att_l):
    mesh = plsc.VectorSubcoreMesh(core_axis_name="c", subcore_axis_name="s")
    return pl.kernel(
        _sc_edge_body,
        out_type=jax.ShapeDtypeStruct((NC, N, ACCW), jnp.float32),
        mesh=mesh,
        compiler_params=pltpu.CompilerParams(use_tc_tiling_on_sc=False,
                                             needs_layout_passes=False),
        scratch_types=[
            pltpu.VMEM((2, G, 3, BLK), jnp.int32),
            pltpu.VMEM((2, C), jnp.float32),
            pltpu.VMEM((2, BLK, HALF), jnp.float32),
            pltpu.VMEM((2, BLK, HALF), jnp.float32),
            pltpu.VMEM((BLK, ACCW), jnp.float32),
            pltpu.VMEM_SHARED((N, ACCW), jnp.float32),
            pltpu.SemaphoreType.DMA,
            pltpu.SemaphoreType.DMA,
        ],
    )(xl2n, xr2n, tab, att_l)


def _ln_rows(t0, t1, lg, lb):
    s = jnp.sum(t0, -1, keepdims=True) + jnp.sum(t1, -1, keepdims=True)
    mu = s / HID
    v = (jnp.sum((t0 - mu) ** 2, -1, keepdims=True)
         + jnp.sum((t1 - mu) ** 2, -1, keepdims=True))
    inv = 1.0 / jnp.sqrt(v / HID + 1e-5)
    y0 = (t0 - mu) * inv * lg[0:1, :] + lb[0:1, :]
    y1 = (t1 - mu) * inv * lg[1:2, :] + lb[1:2, :]
    return y0, y1


def _k0_body(feat, mask, pW, pb, pg, pbeta, semb, Wl0, bl0, Wr0, br0,
             x_out, xl_out, xr_out):
    f = feat[...]
    z = jnp.dot(f, pW[...], preferred_element_type=jnp.float32) + pb[...]
    mu = jnp.mean(z, axis=-1, keepdims=True)
    var = jnp.mean((z - mu) ** 2, axis=-1, keepdims=True)
    xn = (z - mu) / jnp.sqrt(var + 1e-5) * pg[...] + pbeta[...]
    m = mask[...]
    emb = jnp.where(m == 0, semb[0:1, :], semb[1:2, :])
    x = xn + emb
    xl = jnp.dot(x, Wl0[...], preferred_element_type=jnp.float32) + bl0[...]
    xr = jnp.dot(x, Wr0[...], preferred_element_type=jnp.float32) + br0[...]
    x_out[0] = x[:, :HALF]
    x_out[1] = x[:, HALF:]
    xl_out[0] = xl[:, :HALF]
    xl_out[1] = xl[:, HALF:]
    xr_out[0] = xr[:, :HALF]
    xr_out[1] = xr[:, HALF:]


def _combine_halves(acc, x, xl, xr, attbd, hm, cb):
    ts = []
    for q in (0, 1):
        xlq = xl[q]
        xrq = xr[q]
        mm = xlq + xrq
        mm = jnp.maximum(mm, 0.2 * mm)
        a2 = jnp.dot(mm, attbd[q], preferred_element_type=jnp.float32)
        exs = jnp.exp(a2)
        den2 = acc[q][:, 8 * LANES:8 * LANES + 2] + exs
        numer = (acc[q][:, :HALF]
                 + jnp.dot(exs, hm[...], preferred_element_type=jnp.float32)
                 * xlq)
        hq = numer / jnp.dot(den2, hm[...],
                             preferred_element_type=jnp.float32)
        ts.append(hq + cb[q:q + 1, :] + x[q])
    return ts


def _tc_mid_body(acc, x, xl, xr, attbd, hm, cb, lg, lb, Wln, bln, Wrn, brn,
                 x_out, xl_out, xr_out):
    t0, t1 = _combine_halves(acc, x, xl, xr, attbd, hm, cb)
    y0, y1 = _ln_rows(t0, t1, lg, lb)
    y = jnp.concatenate([y0, y1], axis=1)
    x_out[0] = y0
    x_out[1] = y1
    xln = jnp.dot(y, Wln[...], preferred_element_type=jnp.float32) + bln[...]
    xrn = jnp.dot(y, Wrn[...], preferred_element_type=jnp.float32) + brn[...]
    xl_out[0] = xln[:, :HALF]
    xl_out[1] = xln[:, HALF:]
    xr_out[0] = xrn[:, :HALF]
    xr_out[1] = xrn[:, HALF:]


def _tc_fin_body(acc, x, xl, xr, attbd, hm, cb, lg, lb, hW1, hb1, hW2, hb2,
                 o_out):
    t0, t1 = _combine_halves(acc, x, xl, xr, attbd, hm, cb)
    y0, y1 = _ln_rows(t0, t1, lg, lb)
    y = jnp.concatenate([y0, y1], axis=1)
    hh = jnp.dot(y, hW1[...], preferred_element_type=jnp.float32) + hb1[...]
    g = 0.5 * hh * (1.0 + lax.erf(hh / jnp.sqrt(jnp.float32(2.0))))
    o_out[...] = (jnp.dot(g, hW2[...], preferred_element_type=jnp.float32)
                  + hb2[...])


def _full(shape):
    return pl.BlockSpec(shape, lambda i: tuple(0 for _ in shape))


def _rows(shape3):
    return pl.BlockSpec(shape3, lambda i: (0, i, 0))


_GRID = N // ROWS

_k0_call = pl.pallas_call(
    _k0_body,
    grid=(_GRID,),
    in_specs=[
        pl.BlockSpec((ROWS, D), lambda i: (i, 0)),
        pl.BlockSpec((ROWS, 1), lambda i: (i, 0)),
        _full((D, HID)), _full((1, HID)), _full((1, HID)), _full((1, HID)),
        _full((2, HID)),
        _full((HID, HID)), _full((1, HID)),
        _full((HID, HID)), _full((1, HID)),
    ],
    out_specs=[_rows((2, ROWS, HALF))] * 3,
    out_shape=[jax.ShapeDtypeStruct((2, N, HALF), jnp.float32)] * 3,
)

_tc_mid_call = pl.pallas_call(
    _tc_mid_body,
    grid=(_GRID,),
    in_specs=[
        _rows((2, ROWS, ACCW)),
        _rows((2, ROWS, HALF)), _rows((2, ROWS, HALF)), _rows((2, ROWS, HALF)),
        _full((2, HALF, 2)), _full((2, HALF)),
        _full((2, HALF)), _full((2, HALF)), _full((2, HALF)),
        _full((HID, HID)), _full((1, HID)),
        _full((HID, HID)), _full((1, HID)),
    ],
    out_specs=[_rows((2, ROWS, HALF))] * 3,
    out_shape=[jax.ShapeDtypeStruct((2, N, HALF), jnp.float32)] * 3,
)

_tc_fin_call = pl.pallas_call(
    _tc_fin_body,
    grid=(_GRID,),
    in_specs=[
        _rows((2, ROWS, ACCW)),
        _rows((2, ROWS, HALF)), _rows((2, ROWS, HALF)), _rows((2, ROWS, HALF)),
        _full((2, HALF, 2)), _full((2, HALF)),
        _full((2, HALF)), _full((2, HALF)), _full((2, HALF)),
        _full((HID, 128)), _full((1, 128)),
        _full((128, 1)), _full((1, 1)),
    ],
    out_specs=pl.BlockSpec((ROWS, 1), lambda i: (i, 0)),
    out_shape=jax.ShapeDtypeStruct((N, 1), jnp.float32),
)


def kernel(features, edge_index, scale_mask, proj_W, proj_b, proj_g,
           proj_beta, scale_embed, Wl, bl, Wr, br, att, conv_b, ln_g, ln_b,
           head_W1, head_b1, head_W2, head_b2):
    f32 = jnp.float32
    ei = edge_index.astype(jnp.int32)
    src = ei[0].reshape(NS, NB, BLK)
    dst = ei[1].reshape(NS, NB, BLK)
    tab = jnp.stack([
        jnp.stack([src + c * N, dst + c * N, dst], axis=2)
        for c in range(NC)])
    mask2 = scale_mask.astype(jnp.int32).reshape(N, 1)

    def row2(v):
        return v.astype(f32).reshape(1, -1)

    def halves(v):
        return v.astype(f32).reshape(2, HALF)

    hm = jnp.concatenate(
        [jnp.concatenate([jnp.ones((1, C), f32), jnp.zeros((1, C), f32)], 1),
         jnp.concatenate([jnp.zeros((1, C), f32), jnp.ones((1, C), f32)], 1)],
        axis=0)

    x, xl, xr = _k0_call(features.astype(f32), mask2, proj_W.astype(f32),
                         row2(proj_b), row2(proj_g), row2(proj_beta),
                         scale_embed.astype(f32), Wl[0].astype(f32),
                         row2(bl[0]), Wr[0].astype(f32), row2(br[0]))

    out = None
    zc = jnp.zeros((2, C), f32)
    for l in range(L):
        att_l = att[l].astype(f32)
        a_even = jnp.stack([att_l[0], att_l[2]])
        a_odd = jnp.stack([att_l[1], att_l[3]])
        colA = jnp.concatenate([a_even, zc], axis=1)
        colB = jnp.concatenate([zc, a_odd], axis=1)
        attbd = jnp.stack([colA, colB], axis=-1)

        acc = _sc_edge(xl.reshape(2 * N, HALF), xr.reshape(2 * N, HALF),
                       tab, att_l)

        if l < L - 1:
            x, xl, xr = _tc_mid_call(
                acc, x, xl, xr, attbd, hm, halves(conv_b[l]),
                halves(ln_g[l]), halves(ln_b[l]),
                Wl[l + 1].astype(f32), row2(bl[l + 1]),
                Wr[l + 1].astype(f32), row2(br[l + 1]))
        else:
            out = _tc_fin_call(
                acc, x, xl, xr, attbd, hm, halves(conv_b[l]),
                halves(ln_g[l]), halves(ln_b[l]),
                head_W1.astype(f32), row2(head_b1),
                head_W2.astype(f32), head_b2.astype(f32).reshape(1, 1))
    return out[:, 0]


# speedup vs baseline: 29.9139x; 1.0450x over previous
import jax
import jax.numpy as jnp
from jax import lax
from jax.experimental import pallas as pl
from jax.experimental.pallas import tpu as pltpu
from jax.experimental.pallas import tpu_sc as plsc

N, E, D, HID, H, C, L = 10000, 160000, 256, 256, 4, 64, 5
NC, NS, LANES = 2, 16, 16
BLK = 50
NB = E // (NS * BLK)
G = 20
ACCW = 144
ROWS = 1000
HALF = HID // 2


def _sc_edge_body(xl_hbm, xr_hbm, tab_hbm, att_hbm,
                  out_hbm, tabg, att_v, xlg, xrg, wv,
                  acc_sh, sem0, sem1):
    c = lax.axis_index("c")
    t = lax.axis_index("s")
    pltpu.sync_copy(att_hbm.at[pl.ds(2 * c, 2)], att_v)

    zeros = jnp.zeros((LANES,), jnp.float32)

    def zero_row(e, carry):
        for k in range(ACCW // LANES):
            wv[e, pl.ds(k * LANES, LANES)] = zeros
        return carry

    lax.fori_loop(0, BLK, zero_row, 0)
    rpt = N // NS
    nz = rpt // BLK
    for j in range(nz):
        pltpu.sync_copy(wv, acc_sh.at[pl.ds(t * rpt + j * BLK, BLK)])
    rem = rpt - nz * BLK
    if rem:
        pltpu.sync_copy(wv.at[pl.ds(0, rem)],
                        acc_sh.at[pl.ds(t * rpt + nz * BLK, rem)])
    plsc.subcore_barrier()

    att_r = ([att_v[0, pl.ds(k * LANES, LANES)] for k in range(4)]
             + [att_v[1, pl.ds(k * LANES, LANES)] for k in range(4)])
    lane = lax.iota(jnp.int32, LANES)

    def compute_block(slot):
        def edge(e, ecarry):
            xlv = [xlg[slot, e, pl.ds(k * LANES, LANES)] for k in range(8)]
            xrv = [xrg[slot, e, pl.ds(k * LANES, LANES)] for k in range(8)]
            p = []
            for k in range(8):
                mm = xlv[k] + xrv[k]
                mm = jnp.maximum(mm, 0.2 * mm)
                p.append(mm * att_r[k])
            s0 = jnp.sum(p[0] + p[1] + p[2] + p[3])
            s1 = jnp.sum(p[4] + p[5] + p[6] + p[7])
            e0 = jnp.exp(jnp.broadcast_to(s0, (LANES,)))
            e1 = jnp.exp(jnp.broadcast_to(s1, (LANES,)))
            for k in range(4):
                wv[e, pl.ds(k * LANES, LANES)] = xlv[k] * e0
            for k in range(4, 8):
                wv[e, pl.ds(k * LANES, LANES)] = xlv[k] * e1
            den = jnp.where(lane == 0, e0,
                            jnp.where(lane == 1, e1, jnp.zeros_like(e0)))
            wv[e, pl.ds(8 * LANES, LANES)] = den
            return ecarry

        lax.fori_loop(0, BLK, edge, 0, unroll=4)

    def issue(b, slot):
        gs = lax.rem(lax.div(b, G), 2)
        r = lax.rem(b, G)
        cp0 = pltpu.async_copy(xl_hbm.at[tabg.at[gs, r, 0]],
                               xlg.at[slot], sem0)
        cp1 = pltpu.async_copy(xr_hbm.at[tabg.at[gs, r, 1]],
                               xrg.at[slot], sem1)
        return cp0, cp1

    def scatter(b):
        gs = lax.rem(lax.div(b, G), 2)
        r = lax.rem(b, G)
        pltpu.sync_copy(wv, acc_sh.at[tabg.at[gs, r, 2]], add=True)

    pltpu.sync_copy(tab_hbm.at[c, t, pl.ds(0, G)], tabg.at[0])
    issue(0, 0)

    def pair(j, carry):
        b0 = 2 * j
        b1 = b0 + 1
        b2 = b0 + 2
        issue(b1, 1)
        pltpu.make_async_copy(xl_hbm.at[tabg.at[0, 0, 0]], xlg.at[0],
                              sem0).wait()
        pltpu.make_async_copy(xr_hbm.at[tabg.at[0, 0, 1]], xrg.at[0],
                              sem1).wait()
        compute_block(0)

        @pl.when(jnp.logical_and(lax.rem(b2, G) == 0, b2 < NB))
        def _():
            gs2 = lax.rem(lax.div(b2, G), 2)
            pltpu.sync_copy(tab_hbm.at[c, t, pl.ds(b2, G)], tabg.at[gs2])

        @pl.when(b2 < NB)
        def _():
            issue(b2, 0)
        scatter(b0)
        pltpu.make_async_copy(xl_hbm.at[tabg.at[0, 0, 0]], xlg.at[1],
                              sem0).wait()
        pltpu.make_async_copy(xr_hbm.at[tabg.at[0, 0, 1]], xrg.at[1],
                              sem1).wait()
        compute_block(1)
        scatter(b1)
        return carry

    lax.fori_loop(0, NB // 2, pair, 0)
    plsc.subcore_barrier()
    pltpu.sync_copy(acc_sh.at[pl.ds(t * rpt, rpt)],
                    out_hbm.at[c, pl.ds(t * rpt, rpt)])


def _sc_edge(xl2n, xr2n, tab, att_l):
    mesh = plsc.VectorSubcoreMesh(core_axis_name="c", subcore_axis_name="s")
    return pl.kernel(
        _sc_edge_body,
        out_type=jax.ShapeDtypeStruct((NC, N, ACCW), jnp.float32),
        mesh=mesh,
        compiler_params=pltpu.CompilerParams(use_tc_tiling_on_sc=False,
                                             needs_layout_passes=False),
        scratch_types=[
            pltpu.VMEM((2, G, 3, BLK), jnp.int32),
            pltpu.VMEM((2, C), jnp.float32),
            pltpu.VMEM((2, BLK, HALF), jnp.float32),
            pltpu.VMEM((2, BLK, HALF), jnp.float32),
            pltpu.VMEM((BLK, ACCW), jnp.float32),
            pltpu.VMEM_SHARED((N, ACCW), jnp.float32),
            pltpu.SemaphoreType.DMA,
            pltpu.SemaphoreType.DMA,
        ],
    )(xl2n, xr2n, tab, att_l)


def _ln_rows(t0, t1, lg, lb):
    s = jnp.sum(t0, -1, keepdims=True) + jnp.sum(t1, -1, keepdims=True)
    mu = s / HID
    v = (jnp.sum((t0 - mu) ** 2, -1, keepdims=True)
         + jnp.sum((t1 - mu) ** 2, -1, keepdims=True))
    inv = 1.0 / jnp.sqrt(v / HID + 1e-5)
    y0 = (t0 - mu) * inv * lg[0:1, :] + lb[0:1, :]
    y1 = (t1 - mu) * inv * lg[1:2, :] + lb[1:2, :]
    return y0, y1


def _k0_body(feat, mask, pW, pb, pg, pbeta, semb, Wl0, bl0, Wr0, br0,
             x_out, xl_out, xr_out):
    f = feat[...]
    z = jnp.dot(f, pW[...], preferred_element_type=jnp.float32) + pb[...]
    mu = jnp.mean(z, axis=-1, keepdims=True)
    var = jnp.mean((z - mu) ** 2, axis=-1, keepdims=True)
    xn = (z - mu) / jnp.sqrt(var + 1e-5) * pg[...] + pbeta[...]
    m = mask[...]
    emb = jnp.where(m == 0, semb[0:1, :], semb[1:2, :])
    x = xn + emb
    xl = jnp.dot(x, Wl0[...], preferred_element_type=jnp.float32) + bl0[...]
    xr = jnp.dot(x, Wr0[...], preferred_element_type=jnp.float32) + br0[...]
    x_out[0] = x[:, :HALF]
    x_out[1] = x[:, HALF:]
    xl_out[0] = xl[:, :HALF]
    xl_out[1] = xl[:, HALF:]
    xr_out[0] = xr[:, :HALF]
    xr_out[1] = xr[:, HALF:]


def _combine_halves(acc, x, xl, xr, attbd, hm, cb):
    ts = []
    for q in (0, 1):
        xlq = xl[q]
        xrq = xr[q]
        mm = xlq + xrq
        mm = jnp.maximum(mm, 0.2 * mm)
        a2 = jnp.dot(mm, attbd[q], preferred_element_type=jnp.float32)
        exs = jnp.exp(a2)
        den2 = acc[q][:, 8 * LANES:8 * LANES + 2] + exs
        numer = (acc[q][:, :HALF]
                 + jnp.dot(exs, hm[...], preferred_element_type=jnp.float32)
                 * xlq)
        hq = numer / jnp.dot(den2, hm[...],
                             preferred_element_type=jnp.float32)
        ts.append(hq + cb[q:q + 1, :] + x[q])
    return ts


def _tc_mid_body(acc, x, xl, xr, attbd, hm, cb, lg, lb, Wln, bln, Wrn, brn,
                 x_out, xl_out, xr_out):
    t0, t1 = _combine_halves(acc, x, xl, xr, attbd, hm, cb)
    y0, y1 = _ln_rows(t0, t1, lg, lb)
    y = jnp.concatenate([y0, y1], axis=1)
    x_out[0] = y0
    x_out[1] = y1
    xln = jnp.dot(y, Wln[...], preferred_element_type=jnp.float32) + bln[...]
    xrn = jnp.dot(y, Wrn[...], preferred_element_type=jnp.float32) + brn[...]
    xl_out[0] = xln[:, :HALF]
    xl_out[1] = xln[:, HALF:]
    xr_out[0] = xrn[:, :HALF]
    xr_out[1] = xrn[:, HALF:]


def _tc_fin_body(acc, x, xl, xr, attbd, hm, cb, lg, lb, hW1, hb1, hW2, hb2,
                 o_out):
    t0, t1 = _combine_halves(acc, x, xl, xr, attbd, hm, cb)
    y0, y1 = _ln_rows(t0, t1, lg, lb)
    y = jnp.concatenate([y0, y1], axis=1)
    hh = jnp.dot(y, hW1[...], preferred_element_type=jnp.float32) + hb1[...]
    g = 0.5 * hh * (1.0 + lax.erf(hh / jnp.sqrt(jnp.float32(2.0))))
    o_out[...] = (jnp.dot(g, hW2[...], preferred_element_type=jnp.float32)
                  + hb2[...])


def _full(shape):
    return pl.BlockSpec(shape, lambda i: tuple(0 for _ in shape))


def _rows(shape3):
    return pl.BlockSpec(shape3, lambda i: (0, i, 0))


_GRID = N // ROWS

_k0_call = pl.pallas_call(
    _k0_body,
    grid=(_GRID,),
    in_specs=[
        pl.BlockSpec((ROWS, D), lambda i: (i, 0)),
        pl.BlockSpec((ROWS, 1), lambda i: (i, 0)),
        _full((D, HID)), _full((1, HID)), _full((1, HID)), _full((1, HID)),
        _full((2, HID)),
        _full((HID, HID)), _full((1, HID)),
        _full((HID, HID)), _full((1, HID)),
    ],
    out_specs=[_rows((2, ROWS, HALF))] * 3,
    out_shape=[jax.ShapeDtypeStruct((2, N, HALF), jnp.float32)] * 3,
)

_tc_mid_call = pl.pallas_call(
    _tc_mid_body,
    grid=(_GRID,),
    in_specs=[
        _rows((2, ROWS, ACCW)),
        _rows((2, ROWS, HALF)), _rows((2, ROWS, HALF)), _rows((2, ROWS, HALF)),
        _full((2, HALF, 2)), _full((2, HALF)),
        _full((2, HALF)), _full((2, HALF)), _full((2, HALF)),
        _full((HID, HID)), _full((1, HID)),
        _full((HID, HID)), _full((1, HID)),
    ],
    out_specs=[_rows((2, ROWS, HALF))] * 3,
    out_shape=[jax.ShapeDtypeStruct((2, N, HALF), jnp.float32)] * 3,
)

_tc_fin_call = pl.pallas_call(
    _tc_fin_body,
    grid=(_GRID,),
    in_specs=[
        _rows((2, ROWS, ACCW)),
        _rows((2, ROWS, HALF)), _rows((2, ROWS, HALF)), _rows((2, ROWS, HALF)),
        _full((2, HALF, 2)), _full((2, HALF)),
        _full((2, HALF)), _full((2, HALF)), _full((2, HALF)),
        _full((HID, 128)), _full((1, 128)),
        _full((128, 1)), _full((1, 1)),
    ],
    out_specs=pl.BlockSpec((ROWS, 1), lambda i: (i, 0)),
    out_shape=jax.ShapeDtypeStruct((N, 1), jnp.float32),
)


def kernel(features, edge_index, scale_mask, proj_W, proj_b, proj_g,
           proj_beta, scale_embed, Wl, bl, Wr, br, att, conv_b, ln_g, ln_b,
           head_W1, head_b1, head_W2, head_b2):
    f32 = jnp.float32
    ei = edge_index.astype(jnp.int32)
    src = ei[0].reshape(NS, NB, BLK)
    dst = ei[1].reshape(NS, NB, BLK)
    tab = jnp.stack([
        jnp.stack([src + c * N, dst + c * N, dst], axis=2)
        for c in range(NC)])
    mask2 = scale_mask.astype(jnp.int32).reshape(N, 1)

    def row2(v):
        return v.astype(f32).reshape(1, -1)

    def halves(v):
        return v.astype(f32).reshape(2, HALF)

    hm = jnp.concatenate(
        [jnp.concatenate([jnp.ones((1, C), f32), jnp.zeros((1, C), f32)], 1),
         jnp.concatenate([jnp.zeros((1, C), f32), jnp.ones((1, C), f32)], 1)],
        axis=0)

    x, xl, xr = _k0_call(features.astype(f32), mask2, proj_W.astype(f32),
                         row2(proj_b), row2(proj_g), row2(proj_beta),
                         scale_embed.astype(f32), Wl[0].astype(f32),
                         row2(bl[0]), Wr[0].astype(f32), row2(br[0]))

    out = None
    zc = jnp.zeros((2, C), f32)
    for l in range(L):
        att_l = att[l].astype(f32)
        a_even = jnp.stack([att_l[0], att_l[2]])
        a_odd = jnp.stack([att_l[1], att_l[3]])
        colA = jnp.concatenate([a_even, zc], axis=1)
        colB = jnp.concatenate([zc, a_odd], axis=1)
        attbd = jnp.stack([colA, colB], axis=-1)

        acc = _sc_edge(xl.reshape(2 * N, HALF), xr.reshape(2 * N, HALF),
                       tab, att_l)

        if l < L - 1:
            x, xl, xr = _tc_mid_call(
                acc, x, xl, xr, attbd, hm, halves(conv_b[l]),
                halves(ln_g[l]), halves(ln_b[l]),
                Wl[l + 1].astype(f32), row2(bl[l + 1]),
                Wr[l + 1].astype(f32), row2(br[l + 1]))
        else:
            out = _tc_fin_call(
                acc, x, xl, xr, attbd, hm, halves(conv_b[l]),
                halves(ln_g[l]), halves(ln_b[l]),
                head_W1.astype(f32), row2(head_b1),
                head_W2.astype(f32), head_b2.astype(f32).reshape(1, 1))
    return out[:, 0]


# speedup vs baseline: 30.2596x; 1.0116x over previous
import jax
import jax.numpy as jnp
from jax import lax
from jax.experimental import pallas as pl
from jax.experimental.pallas import tpu as pltpu
from jax.experimental.pallas import tpu_sc as plsc

N, E, D, HID, H, C, L = 10000, 160000, 256, 256, 4, 64, 5
NC, NS, LANES = 2, 16, 16
BLK = 50
NB = E // (NS * BLK)
G = 20
ACCW = 144
ROWS = 1000
HALF = HID // 2


def _sc_edge_body(xlr_hbm, gtab_hbm, stab_hbm, att_hbm,
                  out_hbm, gtabg, stabg, att_v, xg, wv,
                  acc_sh, sem0):
    c = lax.axis_index("c")
    t = lax.axis_index("s")
    pltpu.sync_copy(att_hbm.at[pl.ds(2 * c, 2)], att_v)

    zeros = jnp.zeros((LANES,), jnp.float32)

    def zero_row(e, carry):
        for k in range(ACCW // LANES):
            wv[e, pl.ds(k * LANES, LANES)] = zeros
        return carry

    lax.fori_loop(0, BLK, zero_row, 0)
    rpt = N // NS
    nz = rpt // BLK
    for j in range(nz):
        pltpu.sync_copy(wv, acc_sh.at[pl.ds(t * rpt + j * BLK, BLK)])
    rem = rpt - nz * BLK
    if rem:
        pltpu.sync_copy(wv.at[pl.ds(0, rem)],
                        acc_sh.at[pl.ds(t * rpt + nz * BLK, rem)])
    plsc.subcore_barrier()

    att_r = ([att_v[0, pl.ds(k * LANES, LANES)] for k in range(4)]
             + [att_v[1, pl.ds(k * LANES, LANES)] for k in range(4)])
    lane = lax.iota(jnp.int32, LANES)

    def compute_block(slot):
        def edge(e, ecarry):
            xlv = [xg[slot, e, pl.ds(k * LANES, LANES)] for k in range(8)]
            xrv = [xg[slot, BLK + e, pl.ds(k * LANES, LANES)]
                   for k in range(8)]
            p = []
            for k in range(8):
                mm = xlv[k] + xrv[k]
                mm = jnp.maximum(mm, 0.2 * mm)
                p.append(mm * att_r[k])
            s0 = jnp.sum(p[0] + p[1] + p[2] + p[3])
            s1 = jnp.sum(p[4] + p[5] + p[6] + p[7])
            e0 = jnp.exp(jnp.broadcast_to(s0, (LANES,)))
            e1 = jnp.exp(jnp.broadcast_to(s1, (LANES,)))
            for k in range(4):
                wv[e, pl.ds(k * LANES, LANES)] = xlv[k] * e0
            for k in range(4, 8):
                wv[e, pl.ds(k * LANES, LANES)] = xlv[k] * e1
            den = jnp.where(lane == 0, e0,
                            jnp.where(lane == 1, e1, jnp.zeros_like(e0)))
            wv[e, pl.ds(8 * LANES, LANES)] = den
            return ecarry

        lax.fori_loop(0, BLK, edge, 0, unroll=4)

    def issue(b, slot):
        gs = lax.rem(lax.div(b, G), 2)
        r = lax.rem(b, G)
        pltpu.async_copy(xlr_hbm.at[gtabg.at[gs, r]], xg.at[slot], sem0)

    def scatter(b):
        gs = lax.rem(lax.div(b, G), 2)
        r = lax.rem(b, G)
        pltpu.sync_copy(wv, acc_sh.at[stabg.at[gs, r]], add=True)

    def stage_group(b):
        gs = lax.rem(lax.div(b, G), 2)
        pltpu.sync_copy(gtab_hbm.at[c, t, pl.ds(b, G)], gtabg.at[gs])
        pltpu.sync_copy(stab_hbm.at[t, pl.ds(b, G)], stabg.at[gs])

    def drain(slot):
        pltpu.make_async_copy(xlr_hbm.at[gtabg.at[0, 0]], xg.at[slot],
                              sem0).wait()

    stage_group(0)
    issue(0, 0)

    def pair(j, carry):
        b0 = 2 * j
        b1 = b0 + 1
        b2 = b0 + 2
        issue(b1, 1)
        drain(0)
        compute_block(0)

        @pl.when(jnp.logical_and(lax.rem(b2, G) == 0, b2 < NB))
        def _():
            stage_group(b2)

        @pl.when(b2 < NB)
        def _():
            issue(b2, 0)
        scatter(b0)
        drain(1)
        compute_block(1)
        scatter(b1)
        return carry

    lax.fori_loop(0, NB // 2, pair, 0)
    plsc.subcore_barrier()
    pltpu.sync_copy(acc_sh.at[pl.ds(t * rpt, rpt)],
                    out_hbm.at[c, pl.ds(t * rpt, rpt)])


def _sc_edge(xlr, gtab, stab, att_l):
    mesh = plsc.VectorSubcoreMesh(core_axis_name="c", subcore_axis_name="s")
    return pl.kernel(
        _sc_edge_body,
        out_type=jax.ShapeDtypeStruct((NC, N, ACCW), jnp.float32),
        mesh=mesh,
        compiler_params=pltpu.CompilerParams(use_tc_tiling_on_sc=False,
                                             needs_layout_passes=False),
        scratch_types=[
            pltpu.VMEM((2, G, 2 * BLK), jnp.int32),
            pltpu.VMEM((2, G, BLK), jnp.int32),
            pltpu.VMEM((2, C), jnp.float32),
            pltpu.VMEM((2, 2 * BLK, HALF), jnp.float32),
            pltpu.VMEM((BLK, ACCW), jnp.float32),
            pltpu.VMEM_SHARED((N, ACCW), jnp.float32),
            pltpu.SemaphoreType.DMA,
        ],
    )(xlr, gtab, stab, att_l)


def _ln_rows(t0, t1, lg, lb):
    s = jnp.sum(t0, -1, keepdims=True) + jnp.sum(t1, -1, keepdims=True)
    mu = s / HID
    v = (jnp.sum((t0 - mu) ** 2, -1, keepdims=True)
         + jnp.sum((t1 - mu) ** 2, -1, keepdims=True))
    inv = 1.0 / jnp.sqrt(v / HID + 1e-5)
    y0 = (t0 - mu) * inv * lg[0:1, :] + lb[0:1, :]
    y1 = (t1 - mu) * inv * lg[1:2, :] + lb[1:2, :]
    return y0, y1


def _k0_body(feat, mask, pW, pb, pg, pbeta, semb, Wl0, bl0, Wr0, br0,
             x_out, xlr_out):
    f = feat[...]
    z = jnp.dot(f, pW[...], preferred_element_type=jnp.float32) + pb[...]
    mu = jnp.mean(z, axis=-1, keepdims=True)
    var = jnp.mean((z - mu) ** 2, axis=-1, keepdims=True)
    xn = (z - mu) / jnp.sqrt(var + 1e-5) * pg[...] + pbeta[...]
    m = mask[...]
    emb = jnp.where(m == 0, semb[0:1, :], semb[1:2, :])
    x = xn + emb
    xl = jnp.dot(x, Wl0[...], preferred_element_type=jnp.float32) + bl0[...]
    xr = jnp.dot(x, Wr0[...], preferred_element_type=jnp.float32) + br0[...]
    x_out[0] = x[:, :HALF]
    x_out[1] = x[:, HALF:]
    xlr_out[0] = xl[:, :HALF]
    xlr_out[1] = xl[:, HALF:]
    xlr_out[2] = xr[:, :HALF]
    xlr_out[3] = xr[:, HALF:]


def _combine_halves(acc, x, xlr, attbd, hm, cb):
    ts = []
    for q in (0, 1):
        xlq = xlr[q]
        xrq = xlr[2 + q]
        mm = xlq + xrq
        mm = jnp.maximum(mm, 0.2 * mm)
        a2 = jnp.dot(mm, attbd[q], preferred_element_type=jnp.float32)
        exs = jnp.exp(a2)
        den2 = acc[q][:, 8 * LANES:8 * LANES + 2] + exs
        numer = (acc[q][:, :HALF]
                 + jnp.dot(exs, hm[...], preferred_element_type=jnp.float32)
                 * xlq)
        hq = numer / jnp.dot(den2, hm[...],
                             preferred_element_type=jnp.float32)
        ts.append(hq + cb[q:q + 1, :] + x[q])
    return ts


def _tc_mid_body(acc, x, xlr, attbd, hm, cb, lg, lb, Wln, bln, Wrn, brn,
                 x_out, xlr_out):
    t0, t1 = _combine_halves(acc, x, xlr, attbd, hm, cb)
    y0, y1 = _ln_rows(t0, t1, lg, lb)
    y = jnp.concatenate([y0, y1], axis=1)
    x_out[0] = y0
    x_out[1] = y1
    xln = jnp.dot(y, Wln[...], preferred_element_type=jnp.float32) + bln[...]
    xrn = jnp.dot(y, Wrn[...], preferred_element_type=jnp.float32) + brn[...]
    xlr_out[0] = xln[:, :HALF]
    xlr_out[1] = xln[:, HALF:]
    xlr_out[2] = xrn[:, :HALF]
    xlr_out[3] = xrn[:, HALF:]


def _tc_fin_body(acc, x, xlr, attbd, hm, cb, lg, lb, hW1, hb1, hW2, hb2,
                 o_out):
    t0, t1 = _combine_halves(acc, x, xlr, attbd, hm, cb)
    y0, y1 = _ln_rows(t0, t1, lg, lb)
    y = jnp.concatenate([y0, y1], axis=1)
    hh = jnp.dot(y, hW1[...], preferred_element_type=jnp.float32) + hb1[...]
    g = 0.5 * hh * (1.0 + lax.erf(hh / jnp.sqrt(jnp.float32(2.0))))
    o_out[...] = (jnp.dot(g, hW2[...], preferred_element_type=jnp.float32)
                  + hb2[...])


def _full(shape):
    return pl.BlockSpec(shape, lambda i: tuple(0 for _ in shape))


def _rows(shape3):
    return pl.BlockSpec(shape3, lambda i: (0, i, 0))


_GRID = N // ROWS

_k0_call = pl.pallas_call(
    _k0_body,
    grid=(_GRID,),
    in_specs=[
        pl.BlockSpec((ROWS, D), lambda i: (i, 0)),
        pl.BlockSpec((ROWS, 1), lambda i: (i, 0)),
        _full((D, HID)), _full((1, HID)), _full((1, HID)), _full((1, HID)),
        _full((2, HID)),
        _full((HID, HID)), _full((1, HID)),
        _full((HID, HID)), _full((1, HID)),
    ],
    out_specs=[_rows((2, ROWS, HALF)), _rows((4, ROWS, HALF))],
    out_shape=[jax.ShapeDtypeStruct((2, N, HALF), jnp.float32),
               jax.ShapeDtypeStruct((4, N, HALF), jnp.float32)],
)

_tc_mid_call = pl.pallas_call(
    _tc_mid_body,
    grid=(_GRID,),
    in_specs=[
        _rows((2, ROWS, ACCW)),
        _rows((2, ROWS, HALF)), _rows((4, ROWS, HALF)),
        _full((2, HALF, 2)), _full((2, HALF)),
        _full((2, HALF)), _full((2, HALF)), _full((2, HALF)),
        _full((HID, HID)), _full((1, HID)),
        _full((HID, HID)), _full((1, HID)),
    ],
    out_specs=[_rows((2, ROWS, HALF)), _rows((4, ROWS, HALF))],
    out_shape=[jax.ShapeDtypeStruct((2, N, HALF), jnp.float32),
               jax.ShapeDtypeStruct((4, N, HALF), jnp.float32)],
)

_tc_fin_call = pl.pallas_call(
    _tc_fin_body,
    grid=(_GRID,),
    in_specs=[
        _rows((2, ROWS, ACCW)),
        _rows((2, ROWS, HALF)), _rows((4, ROWS, HALF)),
        _full((2, HALF, 2)), _full((2, HALF)),
        _full((2, HALF)), _full((2, HALF)), _full((2, HALF)),
        _full((HID, 128)), _full((1, 128)),
        _full((128, 1)), _full((1, 1)),
    ],
    out_specs=pl.BlockSpec((ROWS, 1), lambda i: (i, 0)),
    out_shape=jax.ShapeDtypeStruct((N, 1), jnp.float32),
)


def kernel(features, edge_index, scale_mask, proj_W, proj_b, proj_g,
           proj_beta, scale_embed, Wl, bl, Wr, br, att, conv_b, ln_g, ln_b,
           head_W1, head_b1, head_W2, head_b2):
    f32 = jnp.float32
    ei = edge_index.astype(jnp.int32)
    src = ei[0].reshape(NS, NB, BLK)
    dst = ei[1].reshape(NS, NB, BLK)
    gtab = jnp.stack([
        jnp.concatenate([src + c * N, dst + (2 + c) * N], axis=2)
        for c in range(NC)])
    stab = dst
    mask2 = scale_mask.astype(jnp.int32).reshape(N, 1)

    def row2(v):
        return v.astype(f32).reshape(1, -1)

    def halves(v):
        return v.astype(f32).reshape(2, HALF)

    hm = jnp.concatenate(
        [jnp.concatenate([jnp.ones((1, C), f32), jnp.zeros((1, C), f32)], 1),
         jnp.concatenate([jnp.zeros((1, C), f32), jnp.ones((1, C), f32)], 1)],
        axis=0)

    x, xlr = _k0_call(features.astype(f32), mask2, proj_W.astype(f32),
                      row2(proj_b), row2(proj_g), row2(proj_beta),
                      scale_embed.astype(f32), Wl[0].astype(f32),
                      row2(bl[0]), Wr[0].astype(f32), row2(br[0]))

    out = None
    zc = jnp.zeros((2, C), f32)
    for l in range(L):
        att_l = att[l].astype(f32)
        a_even = jnp.stack([att_l[0], att_l[2]])
        a_odd = jnp.stack([att_l[1], att_l[3]])
        colA = jnp.concatenate([a_even, zc], axis=1)
        colB = jnp.concatenate([zc, a_odd], axis=1)
        attbd = jnp.stack([colA, colB], axis=-1)

        acc = _sc_edge(xlr.reshape(4 * N, HALF), gtab, stab, att_l)

        if l < L - 1:
            x, xlr = _tc_mid_call(
                acc, x, xlr, attbd, hm, halves(conv_b[l]),
                halves(ln_g[l]), halves(ln_b[l]),
                Wl[l + 1].astype(f32), row2(bl[l + 1]),
                Wr[l + 1].astype(f32), row2(br[l + 1]))
        else:
            out = _tc_fin_call(
                acc, x, xlr, attbd, hm, halves(conv_b[l]),
                halves(ln_g[l]), halves(ln_b[l]),
                head_W1.astype(f32), row2(head_b1),
                head_W2.astype(f32), head_b2.astype(f32).reshape(1, 1))
    return out[:, 0]
